# Initial kernel scaffold; baseline (speedup 1.0000x reference)
import jax
import jax.numpy as jnp
from jax.experimental import pallas as pl


def kernel(x_nchw, w1, b1, w2, b2):
    raise NotImplementedError("write your pallas kernel here")


# speedup vs baseline: 1.2186x; 1.2186x over previous
import functools

import jax
import jax.numpy as jnp
from jax.experimental import pallas as pl
from jax.experimental.pallas import tpu as pltpu


def _se_kernel(x_ref, w1_ref, b1_ref, w2_ref, b2_ref, o_ref, *, inv_hw):
    x = x_ref[...]
    s = jnp.sum(x, axis=-1) * inv_hw
    h = jnp.dot(s, w1_ref[...], preferred_element_type=jnp.float32)
    h = h + b1_ref[...]
    h = jnp.where(h > 0, h, jnp.exp(jnp.minimum(h, 0.0)) - 1.0)
    g = jnp.dot(h, w2_ref[...], preferred_element_type=jnp.float32)
    g = jax.nn.sigmoid(g + b2_ref[...])
    o_ref[...] = x * g[:, :, None]


def kernel(x_nchw, w1, b1, w2, b2):
    B, C, H, W = x_nchw.shape
    HW = H * W
    x = x_nchw.reshape(B, C, HW)
    Cr = w1.shape[1]

    b1r = b1.reshape(1, Cr).astype(jnp.float32)
    b2r = b2.reshape(1, C).astype(jnp.float32)
    w1f = w1.astype(jnp.float32)
    w2f = w2.astype(jnp.float32)

    itemsize = jnp.dtype(x.dtype).itemsize
    out = pl.pallas_call(
        functools.partial(_se_kernel, inv_hw=1.0 / float(HW)),
        out_shape=jax.ShapeDtypeStruct((B, C, HW), x.dtype),
        grid=(B,),
        in_specs=[
            pl.BlockSpec((1, C, HW), lambda b: (b, 0, 0)),
            pl.BlockSpec((C, Cr), lambda b: (0, 0)),
            pl.BlockSpec((1, Cr), lambda b: (0, 0)),
            pl.BlockSpec((Cr, C), lambda b: (0, 0)),
            pl.BlockSpec((1, C), lambda b: (0, 0)),
        ],
        out_specs=pl.BlockSpec((1, C, HW), lambda b: (b, 0, 0)),
        compiler_params=pltpu.CompilerParams(
            dimension_semantics=("parallel",),
            vmem_limit_bytes=100 * 1024 * 1024,
        ),
        cost_estimate=pl.CostEstimate(
            flops=2 * B * C * HW + 4 * B * C * Cr,
            transcendentals=B * C + B * Cr,
            bytes_accessed=2 * x.size * itemsize,
        ),
    )(x, w1f, b1r, w2f, b2r)

    return out.reshape(B, C, H, W)
